# baseline (speedup 1.0000x reference)
import functools

import jax
import jax.numpy as jnp
from jax.experimental import pallas as pl
from jax.experimental.pallas import tpu as pltpu

T, D, E, F, C = 2048, 768, 64, 2048, 128
RC = 32
NRC = C // RC


def _argmax_body(logits_ref, ids_ref):
    ids_ref[...] = jnp.argmax(logits_ref[...], axis=1, keepdims=True).astype(jnp.int32)


def _dispatch_body(ids_ref, tfs_ref, cnt_ref):
    def init_cnt(e, _):
        cnt_ref[e] = 0
        return 0
    jax.lax.fori_loop(0, E, init_cnt, 0, unroll=True)

    def body(t, _):
        e = ids_ref[t]
        p = cnt_ref[e]

        @pl.when(p < C)
        def _():
            tfs_ref[e * C + p] = t

        cnt_ref[e] = p + 1
        return 0
    jax.lax.fori_loop(0, T, body, 0)


def _ffn_body(tfs_ref, cnt_ref, hid_ref, wg_ref, wu_ref, wd_ref, out_ref,
              xb_ref, acc_ref):
    e = pl.program_id(0)
    n = jnp.minimum(cnt_ref[e], C)

    @pl.when(e == 0)
    def _():
        out_ref[...] = jnp.zeros_like(out_ref)

    xb_ref[...] = jnp.zeros_like(xb_ref)

    def gather(c, _):
        t = tfs_ref[e * C + c]
        xb_ref[pl.ds(c, 1), :] = hid_ref[pl.ds(t, 1), :]
        return 0
    jax.lax.fori_loop(0, n, gather, 0)

    for k in range(NRC):
        @pl.when(n > k * RC)
        def _(k=k):
            xs = xb_ref[k * RC:(k + 1) * RC, :]
            g = jnp.dot(xs, wg_ref[0], preferred_element_type=jnp.float32,
                        precision=jax.lax.Precision.DEFAULT)
            u = jnp.dot(xs, wu_ref[0], preferred_element_type=jnp.float32,
                        precision=jax.lax.Precision.DEFAULT)
            h = g * jax.nn.sigmoid(g) * u
            acc_ref[k * RC:(k + 1) * RC, :] = jnp.dot(
                h, wd_ref[0], preferred_element_type=jnp.float32,
                precision=jax.lax.Precision.DEFAULT)

    def scatter(c, _):
        t = tfs_ref[e * C + c]
        out_ref[pl.ds(t, 1), :] = acc_ref[pl.ds(c, 1), :]
        return 0
    jax.lax.fori_loop(0, n, scatter, 0)


@functools.partial(jax.jit, static_argnames=("interpret",))
def kernel(hidden_states, router_logits, w_gate, w_up, w_down, interpret=False):
    ids = pl.pallas_call(
        _argmax_body,
        out_shape=jax.ShapeDtypeStruct((T, 1), jnp.int32),
        interpret=interpret,
    )(router_logits)
    ids = ids.reshape(T)

    tfs, cnt = pl.pallas_call(
        _dispatch_body,
        in_specs=[pl.BlockSpec(memory_space=pltpu.SMEM)],
        out_specs=(pl.BlockSpec(memory_space=pltpu.SMEM),
                   pl.BlockSpec(memory_space=pltpu.SMEM)),
        out_shape=(jax.ShapeDtypeStruct((E * C,), jnp.int32),
                   jax.ShapeDtypeStruct((E,), jnp.int32)),
        interpret=interpret,
    )(ids)

    out = pl.pallas_call(
        _ffn_body,
        grid_spec=pltpu.PrefetchScalarGridSpec(
            num_scalar_prefetch=2,
            grid=(E,),
            in_specs=[
                pl.BlockSpec((T, D), lambda e, *_: (0, 0)),
                pl.BlockSpec((1, D, F), lambda e, *_: (e, 0, 0)),
                pl.BlockSpec((1, D, F), lambda e, *_: (e, 0, 0)),
                pl.BlockSpec((1, F, D), lambda e, *_: (e, 0, 0)),
            ],
            out_specs=pl.BlockSpec((T, D), lambda e, *_: (0, 0)),
            scratch_shapes=[
                pltpu.VMEM((C, D), jnp.float32),
                pltpu.VMEM((C, D), jnp.float32),
            ],
        ),
        out_shape=jax.ShapeDtypeStruct((T, D), jnp.float32),
        interpret=interpret,
    )(tfs, cnt, hidden_states, w_gate, w_up, w_down)
    return out


# speedup vs baseline: 1.0084x; 1.0084x over previous
import functools

import jax
import jax.numpy as jnp
from jax.experimental import pallas as pl
from jax.experimental.pallas import tpu as pltpu

T, D, E, F, C = 2048, 768, 64, 2048, 128
RC = 32
NRC = C // RC


def _argmax_body(logits_ref, ids_ref):
    ids_ref[...] = jnp.argmax(logits_ref[...], axis=1, keepdims=True).astype(jnp.int32)


def _ffn_body(ids_ref, hid_ref, wg_ref, wu_ref, wd_ref, out_ref,
              xb_ref, acc_ref, tfs_ref, cnt_ref):
    e = pl.program_id(0)

    @pl.when(e == 0)
    def _():
        out_ref[...] = jnp.zeros_like(out_ref)

        def init_cnt(i, _):
            cnt_ref[i] = 0
            return 0
        jax.lax.fori_loop(0, E, init_cnt, 0, unroll=True)

        def rank(t, _):
            ex = ids_ref[t]
            p = cnt_ref[ex]

            @pl.when(p < C)
            def _():
                tfs_ref[ex * C + p] = t

            cnt_ref[ex] = p + 1
            return 0
        jax.lax.fori_loop(0, T, rank, 0)

    n = jnp.minimum(cnt_ref[e], C)
    xb_ref[...] = jnp.zeros_like(xb_ref)

    def gather(c, _):
        t = tfs_ref[e * C + c]
        xb_ref[pl.ds(c, 1), :] = hid_ref[pl.ds(t, 1), :]
        return 0
    jax.lax.fori_loop(0, n, gather, 0)

    for k in range(NRC):
        @pl.when(n > k * RC)
        def _(k=k):
            xs = xb_ref[k * RC:(k + 1) * RC, :]
            g = jnp.dot(xs, wg_ref[0], preferred_element_type=jnp.float32)
            u = jnp.dot(xs, wu_ref[0], preferred_element_type=jnp.float32)
            h = g * jax.nn.sigmoid(g) * u
            acc_ref[k * RC:(k + 1) * RC, :] = jnp.dot(
                h, wd_ref[0], preferred_element_type=jnp.float32)

    def scatter(c, _):
        t = tfs_ref[e * C + c]
        out_ref[pl.ds(t, 1), :] = acc_ref[pl.ds(c, 1), :]
        return 0
    jax.lax.fori_loop(0, n, scatter, 0)


@functools.partial(jax.jit, static_argnames=("interpret",))
def kernel(hidden_states, router_logits, w_gate, w_up, w_down, interpret=False):
    ids = pl.pallas_call(
        _argmax_body,
        out_shape=jax.ShapeDtypeStruct((T, 1), jnp.int32),
        interpret=interpret,
    )(router_logits)
    ids = ids.reshape(T)

    out = pl.pallas_call(
        _ffn_body,
        grid_spec=pltpu.PrefetchScalarGridSpec(
            num_scalar_prefetch=1,
            grid=(E,),
            in_specs=[
                pl.BlockSpec((T, D), lambda e, *_: (0, 0)),
                pl.BlockSpec((1, D, F), lambda e, *_: (e, 0, 0)),
                pl.BlockSpec((1, D, F), lambda e, *_: (e, 0, 0)),
                pl.BlockSpec((1, F, D), lambda e, *_: (e, 0, 0)),
            ],
            out_specs=pl.BlockSpec((T, D), lambda e, *_: (0, 0)),
            scratch_shapes=[
                pltpu.VMEM((C, D), jnp.float32),
                pltpu.VMEM((C, D), jnp.float32),
                pltpu.SMEM((E * C,), jnp.int32),
                pltpu.SMEM((E,), jnp.int32),
            ],
        ),
        out_shape=jax.ShapeDtypeStruct((T, D), jnp.float32),
        interpret=interpret,
    )(ids, hidden_states, w_gate, w_up, w_down)
    return out
